# baseline (speedup 1.0000x reference)
import functools

import jax
import jax.numpy as jnp
from jax import lax
from jax.experimental import pallas as pl
from jax.experimental.pallas import tpu as pltpu
from jax.experimental.pallas import tpu_sc as plsc

VOCAB = 1000000
EMBED = 32
SEQ = 50
BATCH = 16384
NUM_CORES = 2
NUM_SUBCORES = 16
DIMS_PER_CORE = EMBED // NUM_CORES
BLK = BATCH // NUM_SUBCORES
ROWS_PER_CHUNK = 5
GCHUNK = ROWS_PER_CHUNK * BLK
N_CHUNKS = SEQ // ROWS_PER_CHUNK


@jax.jit
def _dual_gather(src_t, tgt_t, src_idx_t, tgt_idx_t):
    mesh = plsc.VectorSubcoreMesh(core_axis_name="c", subcore_axis_name="s")

    @functools.partial(
        pl.kernel,
        out_type=(
            jax.ShapeDtypeStruct((SEQ, EMBED, BATCH), jnp.float32),
            jax.ShapeDtypeStruct((SEQ, EMBED, BATCH), jnp.float32),
        ),
        mesh=mesh,
        scratch_types=[
            pltpu.VMEM_SHARED((VOCAB,), jnp.float32),
            pltpu.VMEM((SEQ * BLK,), jnp.int32),
            pltpu.VMEM((GCHUNK,), jnp.float32),
            pltpu.VMEM((GCHUNK,), jnp.float32),
            pltpu.SemaphoreType.DMA,
            pltpu.SemaphoreType.DMA,
        ],
    )
    def body(src_tab, tgt_tab, src_idx, tgt_idx, src_out, tgt_out,
             row_sh, idx_v, g0, g1, sem0, sem1):
        cid = lax.axis_index("c")
        sid = lax.axis_index("s")
        b0 = sid * BLK
        bufs = (g0, g1)
        sems = (sem0, sem1)

        def do_table(tab, idx_hbm, out_hbm):
            def stage_idx(s, carry):
                pltpu.sync_copy(idx_hbm.at[s, pl.ds(b0, BLK)],
                                idx_v.at[pl.ds(s * BLK, BLK)])
                return carry

            lax.fori_loop(0, SEQ, stage_idx, 0)

            def dim_body(dl, carry):
                d = cid * DIMS_PER_CORE + dl

                plsc.subcore_barrier()

                @pl.when(sid == 0)
                def _():
                    pltpu.sync_copy(tab.at[d], row_sh)

                plsc.subcore_barrier()

                def gather(c, b):
                    pltpu.async_copy(
                        row_sh.at[idx_v.at[pl.ds(c * GCHUNK, GCHUNK)]],
                        bufs[b], sems[b])

                def wait_gather(b):
                    pltpu.make_async_copy(
                        row_sh.at[pl.ds(0, GCHUNK)], bufs[b],
                        sems[b]).wait()

                def writeout(c, b):
                    for r in range(ROWS_PER_CHUNK):
                        pltpu.sync_copy(
                            bufs[b].at[pl.ds(r * BLK, BLK)],
                            out_hbm.at[ROWS_PER_CHUNK * c + r, d,
                                       pl.ds(b0, BLK)])

                gather(0, 0)
                gather(1, 1)

                def pair_body(i, c2):
                    c0 = i * 2
                    for b in range(2):
                        c = c0 + b
                        wait_gather(b)
                        writeout(c, b)

                        @pl.when(c + 2 < N_CHUNKS)
                        def _():
                            gather(c + 2, b)

                    return c2

                lax.fori_loop(0, N_CHUNKS // 2, pair_body, 0)
                return carry

            lax.fori_loop(0, DIMS_PER_CORE, dim_body, 0)

        do_table(src_tab, src_idx, src_out)
        do_table(tgt_tab, tgt_idx, tgt_out)

    return body(src_t, tgt_t, src_idx_t, tgt_idx_t)


def kernel(src_table, tgt_table, src_indices, tgt_indices):
    src_out, tgt_out = _dual_gather(
        src_table.T, tgt_table.T, src_indices.T, tgt_indices.T)
    return (jnp.transpose(src_out, (2, 0, 1)),
            jnp.transpose(tgt_out, (2, 0, 1)))
